# Initial kernel scaffold; baseline (speedup 1.0000x reference)
import jax
import jax.numpy as jnp
from jax.experimental import pallas as pl


def kernel(set_features, set_mask, edge_index, query_node_indices, W_embed, b_embed, W_attn, b_attn, W_out, b_out, W_g1, b_g1, W_g2, b_g2, W_m1, b_m1, W_m2, b_m2):
    raise NotImplementedError("write your pallas kernel here")


# speedup vs baseline: 38.9416x; 38.9416x over previous
import functools

import jax
import jax.numpy as jnp
from jax import lax
from jax.experimental import pallas as pl
from jax.experimental.pallas import tpu as pltpu
from jax.experimental.pallas import tpu_sc as plsc

NC = 2
NS = 16
NW = NC * NS
CH = 125
KCD = 8
KCC = 2


def _mesh():
    return plsc.VectorSubcoreMesh(core_axis_name="c", subcore_axis_name="s")


_SC_PARAMS = pltpu.CompilerParams(use_tc_tiling_on_sc=False)


def _deg_body(nwin, rows_per_w, chz, dst2d, zeros1, out, ones_v, idx_v,
              deg_sh, sem):
    c = lax.axis_index("c")
    s = lax.axis_index("s")
    w = c * NS + s
    for i in range(8):
        ones_v[pl.ds(i * 16, 16)] = jnp.ones((16,), jnp.float32)
    pltpu.sync_copy(zeros1.at[pl.ds(s * chz, chz)],
                    deg_sh.at[pl.ds(s * chz, chz)])
    plsc.subcore_barrier()
    row0 = w * rows_per_w

    def win(i, carry):
        r = row0 + i * KCD
        pltpu.sync_copy(dst2d.at[pl.ds(r, KCD)], idx_v)
        ds = [pltpu.async_copy(ones_v.at[pl.ds(0, CH)],
                               deg_sh.at[idx_v.at[j]], sem, add=True)
              for j in range(KCD)]
        for d in ds:
            d.wait()
        return carry

    lax.fori_loop(0, nwin, win, 0)
    plsc.subcore_barrier()
    npp = chz * NS
    pltpu.sync_copy(deg_sh.at[pl.ds(s * chz, chz)],
                    out.at[pl.ds(c * npp + s * chz, chz)])


def _deg_partials(dst2d, zeros1, np_pad):
    nwin = dst2d.shape[0] // (NW * KCD)
    rows_per_w = dst2d.shape[0] // NW
    chz = np_pad // NS
    body = functools.partial(_deg_body, nwin, rows_per_w, chz)
    return pl.kernel(
        body,
        out_type=jax.ShapeDtypeStruct((NC * np_pad,), jnp.float32),
        mesh=_mesh(),
        compiler_params=_SC_PARAMS,
        scratch_types=[
            pltpu.VMEM((128,), jnp.float32),
            pltpu.VMEM((KCD, CH), jnp.int32),
            pltpu.VMEM_SHARED((np_pad,), jnp.float32),
            pltpu.SemaphoreType.DMA(()),
        ],
    )(dst2d, zeros1)


def _conv_body(npairs, rows_per_w, chz, y_hbm, src2d, dst2d, zeros2, out,
               isa, ida, isb, idb, rowsa, rowsb, acc_sh,
               sga, sgb, ssa, ssb):
    c = lax.axis_index("c")
    s = lax.axis_index("s")
    w = c * NS + s
    pltpu.sync_copy(zeros2.at[pl.ds(s * chz, chz)],
                    acc_sh.at[pl.ds(s * chz, chz)])
    plsc.subcore_barrier()
    row0 = w * rows_per_w

    def fire_g(idxref, rowsref, sem):
        for j in range(KCC):
            pltpu.async_copy(y_hbm.at[idxref.at[j]], rowsref.at[j], sem)

    def drain_g(idxref, rowsref, sem):
        for j in range(KCC):
            pltpu.make_async_copy(y_hbm.at[idxref.at[j]], rowsref.at[j],
                                  sem).wait()

    def fire_s(idxref, rowsref, sem):
        for j in range(KCC):
            pltpu.async_copy(rowsref.at[j], acc_sh.at[idxref.at[j]], sem,
                             add=True)

    def drain_s(idxref, rowsref, sem):
        for j in range(KCC):
            pltpu.make_async_copy(rowsref.at[j], acc_sh.at[idxref.at[j]],
                                  sem).wait()

    pltpu.sync_copy(src2d.at[pl.ds(row0, KCC)], isa)
    pltpu.sync_copy(dst2d.at[pl.ds(row0, KCC)], ida)
    fire_g(isa, rowsa, sga)

    def pair(k, carry):
        rb = row0 + (2 * k + 1) * KCC
        rn = row0 + lax.min((2 * k + 2) * KCC, rows_per_w - KCC)
        pltpu.sync_copy(src2d.at[pl.ds(rb, KCC)], isb)
        pltpu.sync_copy(dst2d.at[pl.ds(rb, KCC)], idb)
        fire_g(isb, rowsb, sgb)
        drain_g(isa, rowsa, sga)
        fire_s(ida, rowsa, ssa)
        drain_s(ida, rowsa, ssa)
        pltpu.sync_copy(src2d.at[pl.ds(rn, KCC)], isa)
        pltpu.sync_copy(dst2d.at[pl.ds(rn, KCC)], ida)
        fire_g(isa, rowsa, sga)
        drain_g(isb, rowsb, sgb)
        fire_s(idb, rowsb, ssb)
        drain_s(idb, rowsb, ssb)
        return carry

    lax.fori_loop(0, npairs, pair, 0)
    drain_g(isa, rowsa, sga)
    plsc.subcore_barrier()
    npp = chz * NS
    pltpu.sync_copy(acc_sh.at[pl.ds(s * chz, chz)],
                    out.at[pl.ds(c * npp + s * chz, chz)])


def _conv_partials(y, src2d, dst2d, zeros2, np_pad):
    rows_per_w = src2d.shape[0] // NW
    npairs = rows_per_w // (2 * KCC)
    chz = np_pad // NS
    body = functools.partial(_conv_body, npairs, rows_per_w, chz)
    return pl.kernel(
        body,
        out_type=jax.ShapeDtypeStruct((NC * np_pad, 32), jnp.float32),
        mesh=_mesh(),
        compiler_params=_SC_PARAMS,
        scratch_types=[
            pltpu.VMEM((KCC, CH), jnp.int32),
            pltpu.VMEM((KCC, CH), jnp.int32),
            pltpu.VMEM((KCC, CH), jnp.int32),
            pltpu.VMEM((KCC, CH), jnp.int32),
            pltpu.VMEM((KCC, CH, 32), jnp.float32),
            pltpu.VMEM((KCC, CH, 32), jnp.float32),
            pltpu.VMEM_SHARED((np_pad, 32), jnp.float32),
            pltpu.SemaphoreType.DMA(()),
            pltpu.SemaphoreType.DMA(()),
            pltpu.SemaphoreType.DMA(()),
            pltpu.SemaphoreType.DMA(()),
        ],
    )(y, src2d, dst2d, zeros2)


def _qgather_body(bq, np_pad, p2, y2, dinv, q1d, qb, g0, g1, gy, gq, qv, qv2,
                  r0, r1, ry, rq, sem):
    c = lax.axis_index("c")
    s = lax.axis_index("s")
    w = c * NS + s
    pltpu.sync_copy(q1d, qv)
    pltpu.sync_copy(qb, qv2)
    idx = qv.at[pl.ds(w * bq, bq)]
    idx2 = qv2.at[pl.ds(w * bq, bq)]
    d0 = pltpu.async_copy(p2.at[idx], r0, sem)
    d1 = pltpu.async_copy(p2.at[idx2], r1, sem)
    d2 = pltpu.async_copy(y2.at[idx], ry, sem)
    d3 = pltpu.async_copy(dinv.at[idx], rq, sem)
    d0.wait(); d1.wait(); d2.wait(); d3.wait()
    pltpu.sync_copy(r0, g0.at[pl.ds(w * bq, bq)])
    pltpu.sync_copy(r1, g1.at[pl.ds(w * bq, bq)])
    pltpu.sync_copy(ry, gy.at[pl.ds(w * bq, bq)])
    pltpu.sync_copy(rq, gq.at[pl.ds(w * bq, bq)])


def _qgather(p2, np_pad, y2, dinv, q1d):
    b = q1d.shape[0]
    bq = b // NW
    qb = q1d + np_pad
    body = functools.partial(_qgather_body, bq, np_pad)
    return pl.kernel(
        body,
        out_type=(
            jax.ShapeDtypeStruct((b, 32), jnp.float32),
            jax.ShapeDtypeStruct((b, 32), jnp.float32),
            jax.ShapeDtypeStruct((b, 32), jnp.float32),
            jax.ShapeDtypeStruct((b, 32), jnp.float32),
        ),
        mesh=_mesh(),
        compiler_params=_SC_PARAMS,
        scratch_types=[
            pltpu.VMEM((b,), jnp.int32),
            pltpu.VMEM((b,), jnp.int32),
            pltpu.VMEM((bq, 32), jnp.float32),
            pltpu.VMEM((bq, 32), jnp.float32),
            pltpu.VMEM((bq, 32), jnp.float32),
            pltpu.VMEM((bq, 32), jnp.float32),
            pltpu.SemaphoreType.DMA(()),
        ],
    )(p2, y2, dinv, q1d, qb)


def _pool_body(x_ref, we_ref, be_ref, wa_ref, wo_ref, bo_ref,
               wg1_ref, xw1_ref):
    a = wa_ref[...]
    zs = []
    scs = []
    for si in range(8):
        xs = x_ref[:, si, :]
        z = jnp.maximum(
            jnp.dot(xs, we_ref[...], preferred_element_type=jnp.float32)
            + be_ref[...], 0.0)
        zs.append(z)
        scs.append(jnp.dot(z, a, preferred_element_type=jnp.float32))
    m = scs[0]
    for si in range(1, 8):
        m = jnp.maximum(m, scs[si])
    es = [jnp.exp(sc - m) for sc in scs]
    den = es[0]
    for si in range(1, 8):
        den = den + es[si]
    inv = 1.0 / den
    pooled = zs[0] * (es[0] * inv)
    for si in range(1, 8):
        pooled = pooled + zs[si] * (es[si] * inv)
    wc = jnp.dot(wo_ref[...], wg1_ref[...], preferred_element_type=jnp.float32)
    bc = jnp.dot(bo_ref[...], wg1_ref[...], preferred_element_type=jnp.float32)
    xw1_ref[...] = jnp.dot(pooled, wc,
                           preferred_element_type=jnp.float32) + bc


def _pool(x, we, be, wa, wo, bo, wg1, bn):
    n = x.shape[0]
    grid = n // bn
    return pl.pallas_call(
        _pool_body,
        grid=(grid,),
        in_specs=[
            pl.BlockSpec((bn, 8, 128), lambda i: (i, 0, 0)),
            pl.BlockSpec((128, 64), lambda i: (0, 0)),
            pl.BlockSpec((1, 64), lambda i: (0, 0)),
            pl.BlockSpec((64, 1), lambda i: (0, 0)),
            pl.BlockSpec((64, 32), lambda i: (0, 0)),
            pl.BlockSpec((1, 32), lambda i: (0, 0)),
            pl.BlockSpec((32, 32), lambda i: (0, 0)),
        ],
        out_specs=pl.BlockSpec((bn, 32), lambda i: (i, 0)),
        out_shape=jax.ShapeDtypeStruct((n, 32), jnp.float32),
    )(x, we, be, wa, wo, bo, wg1)


def _scale_body(xw1_ref, p0_ref, p1_ref, y1_ref, dinv_ref):
    deg = p0_ref[...] + p1_ref[...] + 1.0
    dinv = lax.rsqrt(deg)
    y1_ref[...] = xw1_ref[...] * dinv
    dinv_ref[...] = jnp.broadcast_to(dinv, (dinv.shape[0], 32))


def _scale(xw1, p0, p1, bn):
    n = xw1.shape[0]
    grid = n // bn
    return pl.pallas_call(
        _scale_body,
        grid=(grid,),
        in_specs=[
            pl.BlockSpec((bn, 32), lambda i: (i, 0)),
            pl.BlockSpec((bn, 1), lambda i: (i, 0)),
            pl.BlockSpec((bn, 1), lambda i: (i, 0)),
        ],
        out_specs=[
            pl.BlockSpec((bn, 32), lambda i: (i, 0)),
            pl.BlockSpec((bn, 32), lambda i: (i, 0)),
        ],
        out_shape=[
            jax.ShapeDtypeStruct((n, 32), jnp.float32),
            jax.ShapeDtypeStruct((n, 32), jnp.float32),
        ],
    )(xw1, p0, p1)


def _mid_body(p0_ref, p1_ref, y1_ref, dinv_ref, bg1_ref, wg2_ref, y2_ref):
    dinv = dinv_ref[...]
    x1 = jnp.maximum(
        (p0_ref[...] + p1_ref[...] + y1_ref[...]) * dinv + bg1_ref[...], 0.0)
    y2_ref[...] = jnp.dot(x1, wg2_ref[...],
                          preferred_element_type=jnp.float32) * dinv


def _mid(p0, p1, y1, dinv, bg1, wg2, bn):
    n = y1.shape[0]
    grid = n // bn
    return pl.pallas_call(
        _mid_body,
        grid=(grid,),
        in_specs=[
            pl.BlockSpec((bn, 32), lambda i: (i, 0)),
            pl.BlockSpec((bn, 32), lambda i: (i, 0)),
            pl.BlockSpec((bn, 32), lambda i: (i, 0)),
            pl.BlockSpec((bn, 32), lambda i: (i, 0)),
            pl.BlockSpec((1, 32), lambda i: (0, 0)),
            pl.BlockSpec((32, 32), lambda i: (0, 0)),
        ],
        out_specs=pl.BlockSpec((bn, 32), lambda i: (i, 0)),
        out_shape=jax.ShapeDtypeStruct((n, 32), jnp.float32),
    )(p0, p1, y1, dinv, bg1, wg2)


def _final_body(g0_ref, g1_ref, gy_ref, gq_ref, bg2_ref, wm1_ref, bm1_ref,
                wm2_ref, bm2_ref, out_ref):
    x2 = jnp.maximum(
        (g0_ref[...] + g1_ref[...] + gy_ref[...]) * gq_ref[...]
        + bg2_ref[...], 0.0)
    h = jnp.maximum(
        jnp.dot(x2, wm1_ref[...], preferred_element_type=jnp.float32)
        + bm1_ref[...], 0.0)
    out_ref[...] = jnp.dot(h, wm2_ref[...],
                           preferred_element_type=jnp.float32) + bm2_ref[...]


def _final(g0, g1, gy, gq, bg2, wm1, bm1, wm2, bm2):
    b = g0.shape[0]
    return pl.pallas_call(
        _final_body,
        out_shape=jax.ShapeDtypeStruct((b, 1), jnp.float32),
    )(g0, g1, gy, gq, bg2, wm1, bm1, wm2, bm2)


def kernel(set_features, set_mask, edge_index, query_node_indices,
           W_embed, b_embed, W_attn, b_attn, W_out, b_out,
           W_g1, b_g1, W_g2, b_g2, W_m1, b_m1, W_m2, b_m2):
    n = set_features.shape[0]
    e = edge_index.shape[1]
    b = query_node_indices.shape[0]
    np_pad = ((n + (128 * NS) - 1) // (128 * NS)) * (128 * NS)

    src2d = edge_index[0].reshape(e // CH, CH)
    dst2d = edge_index[1].reshape(e // CH, CH)
    zeros1 = jnp.zeros((np_pad,), jnp.float32)
    zeros2 = jnp.zeros((np_pad, 32), jnp.float32)

    degp = _deg_partials(dst2d, zeros1, np_pad)
    p0 = degp[:n, None]
    p1 = degp[np_pad:np_pad + n, None]

    xw1 = _pool(set_features, W_embed, b_embed[None, :],
                W_attn, W_out, b_out[None, :], W_g1, 1000)
    y1, dinv = _scale(xw1, p0, p1, 1000)

    c1 = _conv_partials(y1, src2d, dst2d, zeros2, np_pad)
    y2 = _mid(c1[:n], c1[np_pad:np_pad + n], y1, dinv, b_g1[None, :],
              W_g2, 1000)

    c2 = _conv_partials(y2, src2d, dst2d, zeros2, np_pad)
    g0, g1, gy, gq = _qgather(c2, np_pad, y2, dinv, query_node_indices)

    logits = _final(g0, g1, gy, gq, b_g2[None, :], W_m1, b_m1[None, :],
                    W_m2, b_m2[None, :])
    return logits[:, 0]


# speedup vs baseline: 39.6041x; 1.0170x over previous
import functools

import jax
import jax.numpy as jnp
from jax import lax
from jax.experimental import pallas as pl
from jax.experimental.pallas import tpu as pltpu
from jax.experimental.pallas import tpu_sc as plsc

NC = 2
NS = 16
NW = NC * NS
CH = 125
KCD = 8
KCC = 2


def _mesh():
    return plsc.VectorSubcoreMesh(core_axis_name="c", subcore_axis_name="s")


_SC_PARAMS = pltpu.CompilerParams(use_tc_tiling_on_sc=False)


def _deg_body(nwin, rows_per_w, chz, dst2d, zeros1, out, ones_v, idx_v,
              deg_sh, sem):
    c = lax.axis_index("c")
    s = lax.axis_index("s")
    w = c * NS + s
    for i in range(8):
        ones_v[pl.ds(i * 16, 16)] = jnp.ones((16,), jnp.float32)
    pltpu.sync_copy(zeros1.at[pl.ds(s * chz, chz)],
                    deg_sh.at[pl.ds(s * chz, chz)])
    plsc.subcore_barrier()
    row0 = w * rows_per_w

    def win(i, carry):
        r = row0 + i * KCD
        pltpu.sync_copy(dst2d.at[pl.ds(r, KCD)], idx_v)
        ds = [pltpu.async_copy(ones_v.at[pl.ds(0, CH)],
                               deg_sh.at[idx_v.at[j]], sem, add=True)
              for j in range(KCD)]
        for d in ds:
            d.wait()
        return carry

    lax.fori_loop(0, nwin, win, 0)
    plsc.subcore_barrier()
    npp = chz * NS
    pltpu.sync_copy(deg_sh.at[pl.ds(s * chz, chz)],
                    out.at[pl.ds(c * npp + s * chz, chz)])


def _deg_partials(dst2d, zeros1, np_pad):
    nwin = dst2d.shape[0] // (NW * KCD)
    rows_per_w = dst2d.shape[0] // NW
    chz = np_pad // NS
    body = functools.partial(_deg_body, nwin, rows_per_w, chz)
    return pl.kernel(
        body,
        out_type=jax.ShapeDtypeStruct((NC * np_pad,), jnp.float32),
        mesh=_mesh(),
        compiler_params=_SC_PARAMS,
        scratch_types=[
            pltpu.VMEM((128,), jnp.float32),
            pltpu.VMEM((KCD, CH), jnp.int32),
            pltpu.VMEM_SHARED((np_pad,), jnp.float32),
            pltpu.SemaphoreType.DMA(()),
        ],
    )(dst2d, zeros1)


def _conv_body(npairs, rows_per_w, chz, y_hbm, src2d, dst2d, zeros2, out,
               isa, ida, isb, idb, rowsa, rowsb, acc_sh,
               sga, sgb, ssa, ssb):
    c = lax.axis_index("c")
    s = lax.axis_index("s")
    w = c * NS + s
    pltpu.sync_copy(zeros2.at[pl.ds(s * chz, chz)],
                    acc_sh.at[pl.ds(s * chz, chz)])
    plsc.subcore_barrier()
    row0 = w * rows_per_w

    def fire_g(idxref, rowsref, sem):
        for j in range(KCC):
            pltpu.async_copy(y_hbm.at[idxref.at[j]], rowsref.at[j], sem)

    def drain_g(idxref, rowsref, sem):
        for j in range(KCC):
            pltpu.make_async_copy(y_hbm.at[idxref.at[j]], rowsref.at[j],
                                  sem).wait()

    def fire_s(idxref, rowsref, sem):
        for j in range(KCC):
            pltpu.async_copy(rowsref.at[j], acc_sh.at[idxref.at[j]], sem,
                             add=True)

    def drain_s(idxref, rowsref, sem):
        for j in range(KCC):
            pltpu.make_async_copy(rowsref.at[j], acc_sh.at[idxref.at[j]],
                                  sem).wait()

    pltpu.sync_copy(src2d.at[pl.ds(row0, KCC)], isa)
    pltpu.sync_copy(dst2d.at[pl.ds(row0, KCC)], ida)
    fire_g(isa, rowsa, sga)

    def pair(k, carry):
        rb = row0 + (2 * k + 1) * KCC
        rn = row0 + lax.min((2 * k + 2) * KCC, rows_per_w - KCC)
        pltpu.sync_copy(src2d.at[pl.ds(rb, KCC)], isb)
        pltpu.sync_copy(dst2d.at[pl.ds(rb, KCC)], idb)
        fire_g(isb, rowsb, sgb)
        drain_g(isa, rowsa, sga)
        fire_s(ida, rowsa, ssa)
        drain_s(ida, rowsa, ssa)
        pltpu.sync_copy(src2d.at[pl.ds(rn, KCC)], isa)
        pltpu.sync_copy(dst2d.at[pl.ds(rn, KCC)], ida)
        fire_g(isa, rowsa, sga)
        drain_g(isb, rowsb, sgb)
        fire_s(idb, rowsb, ssb)
        drain_s(idb, rowsb, ssb)
        return carry

    lax.fori_loop(0, npairs, pair, 0)
    drain_g(isa, rowsa, sga)
    plsc.subcore_barrier()
    npp = chz * NS
    pltpu.sync_copy(acc_sh.at[pl.ds(s * chz, chz)],
                    out.at[pl.ds(c * npp + s * chz, chz)])


def _conv_partials(y, src2d, dst2d, zeros2, np_pad):
    rows_per_w = src2d.shape[0] // NW
    npairs = rows_per_w // (2 * KCC)
    chz = np_pad // NS
    body = functools.partial(_conv_body, npairs, rows_per_w, chz)
    return pl.kernel(
        body,
        out_type=jax.ShapeDtypeStruct((NC * np_pad, 32), jnp.float32),
        mesh=_mesh(),
        compiler_params=_SC_PARAMS,
        scratch_types=[
            pltpu.VMEM((KCC, CH), jnp.int32),
            pltpu.VMEM((KCC, CH), jnp.int32),
            pltpu.VMEM((KCC, CH), jnp.int32),
            pltpu.VMEM((KCC, CH), jnp.int32),
            pltpu.VMEM((KCC, CH, 32), jnp.float32),
            pltpu.VMEM((KCC, CH, 32), jnp.float32),
            pltpu.VMEM_SHARED((np_pad, 32), jnp.float32),
            pltpu.SemaphoreType.DMA(()),
            pltpu.SemaphoreType.DMA(()),
            pltpu.SemaphoreType.DMA(()),
            pltpu.SemaphoreType.DMA(()),
        ],
    )(y, src2d, dst2d, zeros2)


def _qgather_body(bq, np_pad, p2, y2, dinv, q1d, qb, g0, g1, gy, gq, qv, qv2,
                  r0, r1, ry, rq, sem):
    c = lax.axis_index("c")
    s = lax.axis_index("s")
    w = c * NS + s
    pltpu.sync_copy(q1d, qv)
    pltpu.sync_copy(qb, qv2)
    idx = qv.at[pl.ds(w * bq, bq)]
    idx2 = qv2.at[pl.ds(w * bq, bq)]
    d0 = pltpu.async_copy(p2.at[idx], r0, sem)
    d1 = pltpu.async_copy(p2.at[idx2], r1, sem)
    d2 = pltpu.async_copy(y2.at[idx], ry, sem)
    d3 = pltpu.async_copy(dinv.at[idx], rq, sem)
    d0.wait(); d1.wait(); d2.wait(); d3.wait()
    pltpu.sync_copy(r0, g0.at[pl.ds(w * bq, bq)])
    pltpu.sync_copy(r1, g1.at[pl.ds(w * bq, bq)])
    pltpu.sync_copy(ry, gy.at[pl.ds(w * bq, bq)])
    pltpu.sync_copy(rq, gq.at[pl.ds(w * bq, bq)])


def _qgather(p2, np_pad, y2, dinv, q1d):
    b = q1d.shape[0]
    bq = b // NW
    qb = q1d + np_pad
    body = functools.partial(_qgather_body, bq, np_pad)
    return pl.kernel(
        body,
        out_type=(
            jax.ShapeDtypeStruct((b, 32), jnp.float32),
            jax.ShapeDtypeStruct((b, 32), jnp.float32),
            jax.ShapeDtypeStruct((b, 32), jnp.float32),
            jax.ShapeDtypeStruct((b, 32), jnp.float32),
        ),
        mesh=_mesh(),
        compiler_params=_SC_PARAMS,
        scratch_types=[
            pltpu.VMEM((b,), jnp.int32),
            pltpu.VMEM((b,), jnp.int32),
            pltpu.VMEM((bq, 32), jnp.float32),
            pltpu.VMEM((bq, 32), jnp.float32),
            pltpu.VMEM((bq, 32), jnp.float32),
            pltpu.VMEM((bq, 32), jnp.float32),
            pltpu.SemaphoreType.DMA(()),
        ],
    )(p2, y2, dinv, q1d, qb)


def _pool_body(x_ref, p0_ref, p1_ref, we_ref, be_ref, wa_ref, wo_ref, bo_ref,
               wg1_ref, y1_ref, dinv_ref):
    a = wa_ref[...]
    zs = []
    scs = []
    for si in range(8):
        xs = x_ref[:, si, :]
        z = jnp.maximum(
            jnp.dot(xs, we_ref[...], preferred_element_type=jnp.float32)
            + be_ref[...], 0.0)
        zs.append(z)
        scs.append(jnp.dot(z, a, preferred_element_type=jnp.float32))
    m = scs[0]
    for si in range(1, 8):
        m = jnp.maximum(m, scs[si])
    es = [jnp.exp(sc - m) for sc in scs]
    den = es[0]
    for si in range(1, 8):
        den = den + es[si]
    inv = 1.0 / den
    pooled = zs[0] * (es[0] * inv)
    for si in range(1, 8):
        pooled = pooled + zs[si] * (es[si] * inv)
    px = jnp.dot(pooled, wo_ref[...],
                 preferred_element_type=jnp.float32) + bo_ref[...]
    xw1 = jnp.dot(px, wg1_ref[...], preferred_element_type=jnp.float32)
    deg = p0_ref[...] + p1_ref[...] + 1.0
    dinv = lax.rsqrt(deg)
    y1_ref[...] = xw1 * dinv
    dinv_ref[...] = jnp.broadcast_to(dinv, (dinv.shape[0], 32))


def _pool(x, p0, p1, we, be, wa, wo, bo, wg1, bn):
    n = x.shape[0]
    grid = n // bn
    return pl.pallas_call(
        _pool_body,
        grid=(grid,),
        in_specs=[
            pl.BlockSpec((bn, 8, 128), lambda i: (i, 0, 0)),
            pl.BlockSpec((bn, 1), lambda i: (i, 0)),
            pl.BlockSpec((bn, 1), lambda i: (i, 0)),
            pl.BlockSpec((128, 64), lambda i: (0, 0)),
            pl.BlockSpec((1, 64), lambda i: (0, 0)),
            pl.BlockSpec((64, 1), lambda i: (0, 0)),
            pl.BlockSpec((64, 32), lambda i: (0, 0)),
            pl.BlockSpec((1, 32), lambda i: (0, 0)),
            pl.BlockSpec((32, 32), lambda i: (0, 0)),
        ],
        out_specs=[
            pl.BlockSpec((bn, 32), lambda i: (i, 0)),
            pl.BlockSpec((bn, 32), lambda i: (i, 0)),
        ],
        out_shape=[
            jax.ShapeDtypeStruct((n, 32), jnp.float32),
            jax.ShapeDtypeStruct((n, 32), jnp.float32),
        ],
    )(x, p0, p1, we, be, wa, wo, bo, wg1)


def _mid_body(p0_ref, p1_ref, y1_ref, dinv_ref, bg1_ref, wg2_ref, y2_ref):
    dinv = dinv_ref[...]
    x1 = jnp.maximum(
        (p0_ref[...] + p1_ref[...] + y1_ref[...]) * dinv + bg1_ref[...], 0.0)
    y2_ref[...] = jnp.dot(x1, wg2_ref[...],
                          preferred_element_type=jnp.float32) * dinv


def _mid(p0, p1, y1, dinv, bg1, wg2, bn):
    n = y1.shape[0]
    grid = n // bn
    return pl.pallas_call(
        _mid_body,
        grid=(grid,),
        in_specs=[
            pl.BlockSpec((bn, 32), lambda i: (i, 0)),
            pl.BlockSpec((bn, 32), lambda i: (i, 0)),
            pl.BlockSpec((bn, 32), lambda i: (i, 0)),
            pl.BlockSpec((bn, 32), lambda i: (i, 0)),
            pl.BlockSpec((1, 32), lambda i: (0, 0)),
            pl.BlockSpec((32, 32), lambda i: (0, 0)),
        ],
        out_specs=pl.BlockSpec((bn, 32), lambda i: (i, 0)),
        out_shape=jax.ShapeDtypeStruct((n, 32), jnp.float32),
    )(p0, p1, y1, dinv, bg1, wg2)


def _final_body(g0_ref, g1_ref, gy_ref, gq_ref, bg2_ref, wm1_ref, bm1_ref,
                wm2_ref, bm2_ref, out_ref):
    x2 = jnp.maximum(
        (g0_ref[...] + g1_ref[...] + gy_ref[...]) * gq_ref[...]
        + bg2_ref[...], 0.0)
    h = jnp.maximum(
        jnp.dot(x2, wm1_ref[...], preferred_element_type=jnp.float32)
        + bm1_ref[...], 0.0)
    out_ref[...] = jnp.dot(h, wm2_ref[...],
                           preferred_element_type=jnp.float32) + bm2_ref[...]


def _final(g0, g1, gy, gq, bg2, wm1, bm1, wm2, bm2):
    b = g0.shape[0]
    return pl.pallas_call(
        _final_body,
        out_shape=jax.ShapeDtypeStruct((b, 1), jnp.float32),
    )(g0, g1, gy, gq, bg2, wm1, bm1, wm2, bm2)


def kernel(set_features, set_mask, edge_index, query_node_indices,
           W_embed, b_embed, W_attn, b_attn, W_out, b_out,
           W_g1, b_g1, W_g2, b_g2, W_m1, b_m1, W_m2, b_m2):
    n = set_features.shape[0]
    e = edge_index.shape[1]
    b = query_node_indices.shape[0]
    np_pad = ((n + (128 * NS) - 1) // (128 * NS)) * (128 * NS)

    src2d = edge_index[0].reshape(e // CH, CH)
    dst2d = edge_index[1].reshape(e // CH, CH)
    zeros1 = jnp.zeros((np_pad,), jnp.float32)
    zeros2 = jnp.zeros((np_pad, 32), jnp.float32)

    degp = _deg_partials(dst2d, zeros1, np_pad)
    p0 = degp[:n, None]
    p1 = degp[np_pad:np_pad + n, None]

    y1, dinv = _pool(set_features, p0, p1, W_embed, b_embed[None, :],
                     W_attn, W_out, b_out[None, :], W_g1, 1000)

    c1 = _conv_partials(y1, src2d, dst2d, zeros2, np_pad)
    y2 = _mid(c1[:n], c1[np_pad:np_pad + n], y1, dinv, b_g1[None, :],
              W_g2, 1000)

    c2 = _conv_partials(y2, src2d, dst2d, zeros2, np_pad)
    g0, g1, gy, gq = _qgather(c2, np_pad, y2, dinv, query_node_indices)

    logits = _final(g0, g1, gy, gq, b_g2[None, :], W_m1, b_m1[None, :],
                    W_m2, b_m2[None, :])
    return logits[:, 0]
